# baseline (speedup 1.0000x reference)
import functools

import jax
import jax.numpy as jnp
from jax import lax
from jax.experimental import pallas as pl
from jax.experimental.pallas import tpu as pltpu
from jax.experimental.pallas import tpu_sc as plsc

M = 100000
D = 128
B = 16384
NC = 2
NS = 16
L = 16
CC = 16
NCH = D // CC
SW = M * CC
OW_PT = (SW // (NS * 128)) * 128
OW_REM_BASE = OW_PT * NS
UW_PT = B * CC // NS
BI = 8192
NBLK = UW_PT // BI


def _scatter_pass(idx_hbm, src_hbm, spmem, idx_vm, src_vm, sem, u0):
    iota = lax.iota(jnp.int32, L)

    def blk_body(blk, carry):
        off = u0 + blk * BI
        d1 = pltpu.async_copy(idx_hbm.at[pl.ds(off, BI)], idx_vm, sem)
        d2 = pltpu.async_copy(src_hbm.at[pl.ds(off, BI)], src_vm, sem)
        d1.wait()
        d2.wait()

        def vec_body(r, c2):
            sl = pl.ds(r * L, L)
            idx_vm[sl] = idx_vm[sl] * CC + iota
            return c2

        lax.fori_loop(0, BI // L, vec_body, 0)

        pltpu.async_copy(src_vm, spmem.at[idx_vm], sem, add=True).wait()
        return carry

    lax.fori_loop(0, NBLK, blk_body, 0)


def _stripe_copy(src, src_base, dst, dst_base, s, o0):
    pltpu.sync_copy(src.at[pl.ds(src_base + o0, OW_PT)],
                    dst.at[pl.ds(dst_base + o0, OW_PT)])

    @pl.when(s < (SW - OW_REM_BASE) // 128)
    def _():
        o2 = OW_REM_BASE + s * 128
        pltpu.sync_copy(src.at[pl.ds(src_base + o2, 128)],
                        dst.at[pl.ds(dst_base + o2, 128)])


def _body(t_hbm, idx_hbm, src_hbm, out_hbm, spmem, idx_vm, src_vm, sem):
    core = lax.axis_index("c")
    s = lax.axis_index("s")
    o0 = s * OW_PT
    u0 = s * UW_PT
    for cc in range(NCH // NC):
        c = core * (NCH // NC) + cc

        _stripe_copy(t_hbm, c * SW, spmem, 0, s, o0)
        plsc.subcore_barrier()

        _scatter_pass(idx_hbm, src_hbm, spmem, idx_vm, src_vm, sem,
                      c * (B * CC) + u0)
        plsc.subcore_barrier()

        _stripe_copy(spmem, 0, out_hbm, c * SW, s, o0)
        plsc.subcore_barrier()


@functools.partial(jax.jit, static_argnums=())
def _scatter_add(t_sw, idx_sw, src_sw):
    f = pl.kernel(
        _body,
        out_type=jax.ShapeDtypeStruct((NCH * SW,), jnp.float32),
        mesh=plsc.VectorSubcoreMesh(core_axis_name="c", subcore_axis_name="s",
                                    num_cores=NC, num_subcores=NS),
        scratch_types=[
            pltpu.VMEM_SHARED((SW,), jnp.float32),
            pltpu.VMEM((BI,), jnp.int32),
            pltpu.VMEM((BI,), jnp.float32),
            pltpu.SemaphoreType.DMA,
        ],
    )
    return f(t_sw, idx_sw, src_sw)


def _swizzle(x, dtype=None):
    n = x.shape[0]
    x = x.reshape(n, NCH, CC).transpose(1, 0, 2).reshape(-1)
    return x if dtype is None else x.astype(dtype)


def kernel(t, dim, index, src):
    del dim
    out_sw = _scatter_add(_swizzle(t), _swizzle(index, jnp.int32),
                          _swizzle(src))
    return out_sw.reshape(NCH, M, CC).transpose(1, 0, 2).reshape(M, D)


# speedup vs baseline: 2.6653x; 2.6653x over previous
import functools

import jax
import jax.numpy as jnp
from jax import lax
from jax.experimental import pallas as pl
from jax.experimental.pallas import tpu as pltpu
from jax.experimental.pallas import tpu_sc as plsc

M = 100000
D = 128
B = 16384
N = B * D
NC = 2
NS = 16
L = 16
CHUNK_ROWS = 12512
NCHUNK = NC * 4
CW = CHUNK_ROWS * D
LAST_ROWS = M - 7 * CHUNK_ROWS
PER_TILE = N // NS
NR = 64
BI = NR * D
TILE_ROWS = B // NS
NBLK = TILE_ROWS // NR


def _copy_rows(src_ref, src_base, dst_ref, dst_base, nrows, s):
    npt = nrows // NS
    w = npt * D
    pltpu.sync_copy(src_ref.at[pl.ds(src_base + s * w, w)],
                    dst_ref.at[pl.ds(dst_base + s * w, w)])


def _scatter_pass(idx_hbm, src_hbm, spmem, idx_vm, src_vm, sem,
                  tile_base, word_lo, span, sid):
    cols = [lax.iota(jnp.int32, L) + (u * L) for u in range(8)]
    uspan = jnp.uint32(span)
    pads = [CW + sid * 64 + ((u * L) & 63) + lax.iota(jnp.int32, L)
            for u in range(8)]

    def blk_body(blk, carry):
        off = tile_base + blk * BI
        d1 = pltpu.async_copy(idx_hbm.at[pl.ds(off, BI)], idx_vm, sem)
        d2 = pltpu.async_copy(src_hbm.at[pl.ds(off, BI)], src_vm, sem)
        d1.wait()
        d2.wait()

        def row_body(r, c2):
            for u in range(8):
                sl = pl.ds(r * 128 + u * L, L)
                idxv = idx_vm[sl]
                srcv = src_vm[sl]
                w = idxv * D + (cols[u] - word_lo)
                m = w.astype(jnp.uint32) < uspan
                idx_vm[sl] = jnp.where(m, w, pads[u])
                src_vm[sl] = jnp.where(m, srcv, jnp.float32(0))
            return c2

        lax.fori_loop(0, BI // 128, row_body, 0)

        pltpu.async_copy(src_vm, spmem.at[idx_vm], sem, add=True).wait()
        return carry

    lax.fori_loop(0, NBLK, blk_body, 0)


def _body(t_hbm, idx_hbm, src_hbm, out_hbm, spmem, idx_vm, src_vm, sem):
    core = lax.axis_index("c")
    s = lax.axis_index("s")
    tile_base = s * PER_TILE
    for cc in range(4):
        chunk = core * 4 + cc
        row_lo = chunk * CHUNK_ROWS
        word_lo = row_lo * D
        is_last = cc == 3

        if not is_last:
            _copy_rows(t_hbm, word_lo, spmem, 0, CHUNK_ROWS, s)
        else:
            @pl.when(core == 0)
            def _():
                _copy_rows(t_hbm, word_lo, spmem, 0, CHUNK_ROWS, s)

            @pl.when(core == 1)
            def _():
                _copy_rows(t_hbm, word_lo, spmem, 0, LAST_ROWS, s)

        plsc.subcore_barrier()

        if not is_last:
            _scatter_pass(idx_hbm, src_hbm, spmem, idx_vm, src_vm, sem,
                          tile_base, word_lo, CW, s)
        else:
            @pl.when(core == 0)
            def _():
                _scatter_pass(idx_hbm, src_hbm, spmem, idx_vm, src_vm, sem,
                              tile_base, word_lo, CW, s)

            @pl.when(core == 1)
            def _():
                _scatter_pass(idx_hbm, src_hbm, spmem, idx_vm, src_vm, sem,
                              tile_base, word_lo, LAST_ROWS * D, s)

        plsc.subcore_barrier()

        if not is_last:
            _copy_rows(spmem, 0, out_hbm, word_lo, CHUNK_ROWS, s)
        else:
            @pl.when(core == 0)
            def _():
                _copy_rows(spmem, 0, out_hbm, word_lo, CHUNK_ROWS, s)

            @pl.when(core == 1)
            def _():
                _copy_rows(spmem, 0, out_hbm, word_lo, LAST_ROWS, s)

        plsc.subcore_barrier()


@functools.partial(jax.jit, static_argnums=())
def _scatter_add_flat(t_flat, idx_flat, src_flat):
    f = pl.kernel(
        _body,
        out_type=jax.ShapeDtypeStruct((M * D,), jnp.float32),
        mesh=plsc.VectorSubcoreMesh(core_axis_name="c", subcore_axis_name="s",
                                    num_cores=NC, num_subcores=NS),
        scratch_types=[
            pltpu.VMEM_SHARED((CW + 1024,), jnp.float32),
            pltpu.VMEM((BI,), jnp.int32),
            pltpu.VMEM((BI,), jnp.float32),
            pltpu.SemaphoreType.DMA,
        ],
    )
    return f(t_flat, idx_flat, src_flat)


def kernel(t, dim, index, src):
    del dim
    out = _scatter_add_flat(t.reshape(-1), index.astype(jnp.int32).reshape(-1),
                            src.reshape(-1))
    return out.reshape(t.shape)
